# Initial kernel scaffold; baseline (speedup 1.0000x reference)
import jax
import jax.numpy as jnp
from jax.experimental import pallas as pl


def kernel(logits):
    raise NotImplementedError("write your pallas kernel here")


# speedup vs baseline: 263.4865x; 263.4865x over previous
import functools

import numpy as np
import jax
import jax.numpy as jnp
from jax import lax
from jax.experimental import pallas as pl
from jax.experimental.pallas import tpu as pltpu

_KSUB = 10
_NEG = np.float32(-300.0)
_CLIP = np.float32(-1e-7)
_LN2 = np.float32(-0.6931471805599453)


def _rotl(x, r):
    return ((x << np.uint32(r)) | (x >> np.uint32(32 - r))).astype(np.uint32)


def _threefry2x32(k0, k1, x0, x1):
    rotations = (13, 15, 26, 6, 17, 29, 16, 24)
    ks0 = np.uint32(k0)
    ks1 = np.uint32(k1)
    ks2 = np.uint32(ks0 ^ ks1 ^ np.uint32(0x1BD11BDA))
    ks = (ks0, ks1, ks2)
    x0 = (x0 + ks0).astype(np.uint32)
    x1 = (x1 + ks1).astype(np.uint32)
    for i in range(5):
        rots = rotations[0:4] if i % 2 == 0 else rotations[4:8]
        for r in rots:
            x0 = (x0 + x1).astype(np.uint32)
            x1 = _rotl(x1, r)
            x1 = (x1 ^ x0).astype(np.uint32)
        x0 = (x0 + ks[(i + 1) % 3]).astype(np.uint32)
        x1 = (x1 + ks[(i + 2) % 3] + np.uint32(i + 1)).astype(np.uint32)
    return x0, x1


@functools.lru_cache(maxsize=None)
def _uniforms_by_pos(n, b):
    key = np.array([0, 42], dtype=np.uint32)
    zeros2 = np.zeros(2, np.uint32)
    count2 = np.arange(2, dtype=np.uint32)
    zerosb = np.zeros(b, np.uint32)
    countb = np.arange(b, dtype=np.uint32)
    out = np.empty((n, b), dtype=np.float32)
    for t in range(n):
        o0, o1 = _threefry2x32(key[0], key[1], zeros2, count2)
        key = np.array([o0[0], o1[0]], np.uint32)
        s0, s1 = _threefry2x32(o0[1], o1[1], zerosb, countb)
        bits = (s0 ^ s1).astype(np.uint32)
        fb = ((bits >> np.uint32(9)) | np.uint32(0x3F800000)).astype(np.uint32)
        out[t] = fb.view(np.float32) - np.float32(1.0)
    return np.flipud(out).copy()


def _log1mexp(x):
    big = x > _LN2
    x1 = jnp.where(big, x, _LN2)
    x2 = jnp.where(big, _LN2, x)
    em1 = jnp.exp(x1) - np.float32(1.0)
    return jnp.where(big, jnp.log(-em1), jnp.log1p(-jnp.exp(x2)))


def _logaddexp(x1, x2):
    amax = jnp.maximum(x1, x2)
    d = x1 - x2
    return amax + jnp.log1p(jnp.exp(-jnp.abs(d)))


def _body(theta_ref, u_ref, out_ref, a_ref, lp_ref, lq_ref):
    n, bb = theta_ref.shape
    kk2 = _KSUB + 2

    th = theta_ref[...]
    negth = -th
    softplus = jnp.maximum(negth, np.float32(0.0)) + jnp.log1p(
        jnp.exp(-jnp.abs(negth))
    )
    lp = jnp.minimum(-softplus, _CLIP)
    lq = _log1mexp(lp)
    lp_ref[...] = lp
    lq_ref[...] = lq

    iota_k = lax.broadcasted_iota(jnp.int32, (kk2, bb), 0)
    state0 = jnp.where(iota_k == 1, np.float32(0.0), _NEG)
    a_ref[0] = state0

    def dp_step(i, state):
        lp_i = lp_ref[pl.ds(i, 1), :]
        lq_i = lq_ref[pl.ds(i, 1), :]
        new = _logaddexp(state[: kk2 - 1] + lp_i, state[1:] + lq_i)
        state = jnp.concatenate(
            [jnp.full((1, bb), _NEG, jnp.float32), new], axis=0
        )
        a_ref[i + 1] = state
        return state

    lax.fori_loop(0, n, dp_step, state0)

    def s_step(t, j):
        i = n - t
        a_prev = a_ref[i - 1]
        a_cur = a_ref[i]
        mp = iota_k == jnp.maximum(j - 1, 0)
        mz = iota_k == jnp.maximum(j, 0)
        p = jnp.sum(jnp.where(mp, a_prev, np.float32(0.0)), axis=0, keepdims=True)
        z = jnp.sum(jnp.where(mz, a_cur, np.float32(0.0)), axis=0, keepdims=True)
        lp_i = lp_ref[pl.ds(i - 1, 1), :]
        praw = jnp.minimum(p + lp_i - z, _CLIP)
        q = _log1mexp(praw)
        prob = jax.nn.sigmoid(praw - q)
        u = u_ref[pl.ds(i - 1, 1), :]
        xb = u < prob
        out_ref[pl.ds(i - 1, 1), :] = jnp.where(
            xb, np.float32(1.0), np.float32(0.0)
        )
        return jnp.where(xb, j - 1, j)

    j0 = jnp.full((1, bb), kk2 - 1, jnp.int32)
    lax.fori_loop(0, n, s_step, j0)


def _sample_call(theta_t, u):
    n, b = theta_t.shape
    kk2 = _KSUB + 2
    return pl.pallas_call(
        _body,
        out_shape=jax.ShapeDtypeStruct((n, b), jnp.float32),
        scratch_shapes=[
            pltpu.VMEM((n + 1, kk2, b), jnp.float32),
            pltpu.VMEM((n, b), jnp.float32),
            pltpu.VMEM((n, b), jnp.float32),
        ],
    )(theta_t, u)


def kernel(logits):
    theta = jnp.squeeze(logits, -1)
    b, n = theta.shape
    u = jnp.asarray(_uniforms_by_pos(n, b))
    samples_t = _sample_call(theta.T, u)
    return samples_t.T[..., None]


# speedup vs baseline: 402.3309x; 1.5270x over previous
import functools

import numpy as np
import jax
import jax.numpy as jnp
from jax import lax
from jax.experimental import pallas as pl
from jax.experimental.pallas import tpu as pltpu

_KSUB = 10
_NEG = np.float32(-300.0)
_CLIP = np.float32(-1e-7)
_LN2 = np.float32(-0.6931471805599453)


def _rotl(x, r):
    return ((x << np.uint32(r)) | (x >> np.uint32(32 - r))).astype(np.uint32)


def _threefry2x32(k0, k1, x0, x1):
    rotations = (13, 15, 26, 6, 17, 29, 16, 24)
    ks0 = np.uint32(k0)
    ks1 = np.uint32(k1)
    ks2 = np.uint32(ks0 ^ ks1 ^ np.uint32(0x1BD11BDA))
    ks = (ks0, ks1, ks2)
    x0 = (x0 + ks0).astype(np.uint32)
    x1 = (x1 + ks1).astype(np.uint32)
    for i in range(5):
        rots = rotations[0:4] if i % 2 == 0 else rotations[4:8]
        for r in rots:
            x0 = (x0 + x1).astype(np.uint32)
            x1 = _rotl(x1, r)
            x1 = (x1 ^ x0).astype(np.uint32)
        x0 = (x0 + ks[(i + 1) % 3]).astype(np.uint32)
        x1 = (x1 + ks[(i + 2) % 3] + np.uint32(i + 1)).astype(np.uint32)
    return x0, x1


@functools.lru_cache(maxsize=None)
def _uniforms_by_pos(n, b):
    key = np.array([0, 42], dtype=np.uint32)
    zeros2 = np.zeros(2, np.uint32)
    count2 = np.arange(2, dtype=np.uint32)
    zerosb = np.zeros(b, np.uint32)
    countb = np.arange(b, dtype=np.uint32)
    out = np.empty((n, b), dtype=np.float32)
    for t in range(n):
        o0, o1 = _threefry2x32(key[0], key[1], zeros2, count2)
        key = np.array([o0[0], o1[0]], np.uint32)
        s0, s1 = _threefry2x32(o0[1], o1[1], zerosb, countb)
        bits = (s0 ^ s1).astype(np.uint32)
        fb = ((bits >> np.uint32(9)) | np.uint32(0x3F800000)).astype(np.uint32)
        out[t] = fb.view(np.float32) - np.float32(1.0)
    return np.flipud(out).copy()


def _log1mexp(x):
    big = x > _LN2
    x1 = jnp.where(big, x, _LN2)
    x2 = jnp.where(big, _LN2, x)
    em1 = jnp.exp(x1) - np.float32(1.0)
    return jnp.where(big, jnp.log(-em1), jnp.log1p(-jnp.exp(x2)))


def _logaddexp(x1, x2):
    amax = jnp.maximum(x1, x2)
    d = x1 - x2
    return amax + jnp.log1p(jnp.exp(-jnp.abs(d)))


def _body(theta_ref, u_ref, out_ref, a_ref, lp_ref, lq_ref, t_ref):
    n, bb = theta_ref.shape
    kk2 = _KSUB + 2

    th = theta_ref[...]
    negth = -th
    softplus = jnp.maximum(negth, np.float32(0.0)) + jnp.log1p(
        jnp.exp(-jnp.abs(negth))
    )
    lp = jnp.minimum(-softplus, _CLIP)
    lq = _log1mexp(lp)
    lp_ref[...] = lp
    lq_ref[...] = lq

    iota_k = lax.broadcasted_iota(jnp.int32, (kk2, bb), 0)
    state0 = jnp.where(iota_k == 1, np.float32(0.0), _NEG)
    a_ref[0] = state0

    def dp_step(i, state):
        lp_i = lp_ref[pl.ds(i, 1), :]
        lq_i = lq_ref[pl.ds(i, 1), :]
        new = _logaddexp(state[: kk2 - 1] + lp_i, state[1:] + lq_i)
        state = jnp.concatenate(
            [jnp.full((1, bb), _NEG, jnp.float32), new], axis=0
        )
        a_ref[i + 1] = state
        return state

    lax.fori_loop(0, n, dp_step, state0)

    a_prev = a_ref[0:n]
    a_cur = a_ref[1 : n + 1]
    a_prev_sh = jnp.concatenate(
        [a_prev[:, 0:1, :], a_prev[:, 0 : kk2 - 1, :]], axis=1
    )
    lp3 = lp_ref[...].reshape(n, 1, bb)
    praw = jnp.minimum((a_prev_sh + lp3) - a_cur, _CLIP)
    q = _log1mexp(praw)
    t_ref[...] = jax.nn.sigmoid(praw - q)

    def s_step(t, j):
        m = n - 1 - t
        probrow = t_ref[m]
        mz = iota_k == jnp.maximum(j, 0)
        prob = jnp.sum(
            jnp.where(mz, probrow, np.float32(0.0)), axis=0, keepdims=True
        )
        xb = u_ref[pl.ds(m, 1), :] < prob
        out_ref[pl.ds(m, 1), :] = jnp.where(
            xb, np.float32(1.0), np.float32(0.0)
        )
        return jnp.where(xb, j - 1, j)

    j0 = jnp.full((1, bb), kk2 - 1, jnp.int32)
    lax.fori_loop(0, n, s_step, j0)


def _sample_call(theta_t, u):
    n, b = theta_t.shape
    kk2 = _KSUB + 2
    return pl.pallas_call(
        _body,
        out_shape=jax.ShapeDtypeStruct((n, b), jnp.float32),
        scratch_shapes=[
            pltpu.VMEM((n + 1, kk2, b), jnp.float32),
            pltpu.VMEM((n, b), jnp.float32),
            pltpu.VMEM((n, b), jnp.float32),
            pltpu.VMEM((n, kk2, b), jnp.float32),
        ],
    )(theta_t, u)


def kernel(logits):
    theta = jnp.squeeze(logits, -1)
    b, n = theta.shape
    u = jnp.asarray(_uniforms_by_pos(n, b))
    samples_t = _sample_call(theta.T, u)
    return samples_t.T[..., None]
